# baseline (speedup 1.0000x reference)
import functools

import jax
import jax.numpy as jnp
from jax import lax
from jax.experimental import pallas as pl
from jax.experimental.pallas import tpu as pltpu
from jax.experimental.pallas import tpu_sc as plsc

_NUM_FIELDS = 26
_VOCAB = 100000
_D = 16
_BATCH = 16384
_NSPLIT = 2
_FIELDS_H = _NUM_FIELDS // _NSPLIT
_TOTAL_H = _BATCH * _FIELDS_H
_NW = 32
_PER_W = _TOTAL_H // _NW
_CHUNK = 1664
_NCHUNK = _PER_W // _CHUNK


def _build():
    mesh = plsc.VectorSubcoreMesh(core_axis_name="c", subcore_axis_name="s")

    @functools.partial(
        pl.kernel,
        mesh=mesh,
        out_type=jax.ShapeDtypeStruct((_TOTAL_H, _D), jnp.float32),
        compiler_params=pltpu.CompilerParams(use_tc_tiling_on_sc=False),
        scratch_types=[
            pltpu.VMEM((_PER_W,), jnp.int32),
            pltpu.VMEM((2, _CHUNK, _D), jnp.float32),
            pltpu.SemaphoreType.DMA,
            pltpu.SemaphoreType.DMA,
        ],
    )
    def emb(idx_hbm, table_hbm, out_hbm, idx_v, rows_v, sem0, sem1):
        sems = (sem0, sem1)
        wid = lax.axis_index("s") * 2 + lax.axis_index("c")
        base = wid * _PER_W
        pltpu.sync_copy(idx_hbm.at[pl.ds(base, _PER_W)], idx_v)

        def gather(j, slot):
            return pltpu.async_copy(
                table_hbm.at[idx_v.at[pl.ds(j * _CHUNK, _CHUNK)]],
                rows_v.at[slot],
                sems[slot],
            )

        cps = [None, None]
        cps[0] = gather(0, 0)
        for j in range(_NCHUNK):
            slot = j % 2
            if j + 1 < _NCHUNK:
                cps[1 - slot] = gather(j + 1, 1 - slot)
            cps[slot].wait()
            pltpu.sync_copy(
                rows_v.at[slot], out_hbm.at[pl.ds(base + j * _CHUNK, _CHUNK)]
            )

    return emb


_emb_lookup = _build()


def kernel(x_cat, tables):
    offs = jnp.arange(_FIELDS_H, dtype=jnp.int32) * _VOCAB
    halves = []
    for h in range(_NSPLIT):
        f0 = h * _FIELDS_H
        flat_idx = (x_cat[:, f0:f0 + _FIELDS_H] + offs[None, :]).T.reshape(
            _TOTAL_H)
        flat_tables = tables[f0:f0 + _FIELDS_H].reshape(
            _FIELDS_H * _VOCAB, _D)
        out_h = _emb_lookup(flat_idx, flat_tables)
        halves.append(out_h.reshape(_FIELDS_H, _BATCH, _D))
    return jnp.concatenate(halves, axis=0).transpose(1, 0, 2)


# speedup vs baseline: 1.4390x; 1.4390x over previous
import functools

import jax
import jax.numpy as jnp
from jax import lax
from jax.experimental import pallas as pl
from jax.experimental.pallas import tpu as pltpu
from jax.experimental.pallas import tpu_sc as plsc

_NUM_FIELDS = 26
_VOCAB = 100000
_D = 16
_BATCH = 16384
_TOTAL = _BATCH * _NUM_FIELDS
_NW = 32
_PER_W = _TOTAL // _NW
_CHUNK = 1664
_NCHUNK = _PER_W // _CHUNK
_LANES = 16


def _build():
    mesh = plsc.VectorSubcoreMesh(core_axis_name="c", subcore_axis_name="s")

    @functools.partial(
        pl.kernel,
        mesh=mesh,
        out_type=jax.ShapeDtypeStruct((_TOTAL, _D), jnp.float32),
        compiler_params=pltpu.CompilerParams(use_tc_tiling_on_sc=False),
        scratch_types=[
            pltpu.VMEM((_PER_W,), jnp.int32),
            pltpu.VMEM((2, _CHUNK, _D), jnp.float32),
            pltpu.SemaphoreType.DMA,
            pltpu.SemaphoreType.DMA,
        ],
    )
    def emb(xcat_hbm, table_hbm, out_hbm, idx_v, rows_v, sem0, sem1):
        sems = (sem0, sem1)
        wid = lax.axis_index("s") * 2 + lax.axis_index("c")
        base = wid * _PER_W
        pltpu.sync_copy(xcat_hbm.at[pl.ds(base, _PER_W)], idx_v)

        def gather(j, slot):
            return pltpu.async_copy(
                table_hbm.at[idx_v.at[pl.ds(j * _CHUNK, _CHUNK)]],
                rows_v.at[slot],
                sems[slot],
            )

        cps = [None, None]
        cps[0] = gather(0, 0)
        for j in range(_NCHUNK):
            slot = j % 2
            if j + 1 < _NCHUNK:
                cps[1 - slot] = gather(j + 1, 1 - slot)
            cps[slot].wait()
            pltpu.sync_copy(
                rows_v.at[slot], out_hbm.at[pl.ds(base + j * _CHUNK, _CHUNK)]
            )

    return emb


_emb_lookup = _build()


def kernel(x_cat, tables):
    offs = jnp.arange(_NUM_FIELDS, dtype=jnp.int32) * _VOCAB
    flat_idx = (x_cat + offs[None, :]).T.reshape(_TOTAL)
    flat_tables = tables.reshape(_NUM_FIELDS * _VOCAB, _D)
    out = _emb_lookup(flat_idx, flat_tables)
    return out.reshape(_NUM_FIELDS, _BATCH, _D).transpose(1, 0, 2)
